# baseline (speedup 1.0000x reference)
import dataclasses

import jax
import jax.numpy as jnp
from jax import lax
from jax.experimental import pallas as pl
from jax.experimental.pallas import tpu as pltpu
from jax.experimental.pallas import tpu_sc as plsc

_CP = pltpu.CompilerParams()
if "needs_layout_passes" in pltpu.CompilerParams.__dataclass_fields__:
    _CP = dataclasses.replace(_CP, needs_layout_passes=False)

_NUM_CORES = 2
_NUM_SUBCORES = 16
_NUM_WORKERS = _NUM_CORES * _NUM_SUBCORES
_W = 400
_LANES = 16
_VB = 2048


_HALF = 512000


def _fuse_table(table):
    vocab, hidden = table.shape
    tt = table.T
    nblk = _HALF // _VB
    last_blk = (vocab - 1) // _VB

    def body(t1_ref, t2_ref, out_ref):
        out_ref[...] = jnp.concatenate(
            [t1_ref[...].T, t2_ref[...].T], axis=1
        )

    return pl.pallas_call(
        body,
        grid=(nblk,),
        in_specs=[
            pl.BlockSpec((hidden, _VB), lambda i: (0, i)),
            pl.BlockSpec(
                (hidden, _VB), lambda i: (0, jnp.minimum(nblk + i, last_blk))
            ),
        ],
        out_specs=pl.BlockSpec((_VB, 2 * hidden), lambda i: (i, 0)),
        out_shape=jax.ShapeDtypeStruct((_HALF, 2 * hidden), table.dtype),
    )(tt, tt)


def kernel(input_ids, table):
    batch, seq = input_ids.shape
    vocab, hidden = table.shape
    tabf = _fuse_table(table)
    ids = input_ids.reshape(n_flat := batch * seq)
    in_left = ids < _HALF
    fused = jnp.where(in_left, ids, ids - _HALF).astype(jnp.int32)

    per_worker = n_flat // _NUM_WORKERS
    steps = per_worker // _W
    assert per_worker % _W == 0 and n_flat % _NUM_WORKERS == 0

    mesh = plsc.VectorSubcoreMesh(core_axis_name="c", subcore_axis_name="s")

    @pl.kernel(
        out_type=jax.ShapeDtypeStruct((n_flat, 2 * hidden), table.dtype),
        mesh=mesh,
        scratch_types=[
            pltpu.VMEM((_W,), jnp.int32),
            pltpu.VMEM((_W,), jnp.int32),
            pltpu.VMEM((_W, 2 * hidden), table.dtype),
            pltpu.VMEM((_W, 2 * hidden), table.dtype),
            pltpu.SemaphoreType.DMA,
            pltpu.SemaphoreType.DMA,
            pltpu.SemaphoreType.DMA,
            pltpu.SemaphoreType.DMA,
        ],
        compiler_params=_CP,
    )
    def gather_kernel(tab_hbm, fused_hbm, out_hbm, fidx0, fidx1,
                      rows0, rows1, gsem0, gsem1, osem0, osem1):
        wid = lax.axis_index("s") * _NUM_CORES + lax.axis_index("c")
        base = wid * per_worker
        fidx = (fidx0, fidx1)
        rows = (rows0, rows1)
        gsem = (gsem0, gsem1)
        osem = (osem0, osem1)

        def start_gather(s, slot, not_first):
            off = base + s * _W
            poff = base + (s - 2) * _W

            @pl.when(not_first)
            def _():
                pltpu.make_async_copy(
                    rows[slot], out_hbm.at[pl.ds(poff, _W)], osem[slot]
                ).wait()

            pltpu.sync_copy(fused_hbm.at[pl.ds(off, _W)], fidx[slot])
            pltpu.make_async_copy(
                tab_hbm.at[fidx[slot]], rows[slot], gsem[slot]
            ).start()

        def process(s, slot):
            pltpu.make_async_copy(
                tab_hbm.at[fidx[slot]], rows[slot], gsem[slot]
            ).wait()
            off = base + s * _W
            pltpu.make_async_copy(
                rows[slot], out_hbm.at[pl.ds(off, _W)], osem[slot]
            ).start()

        start_gather(0, 0, not_first=False)

        @pl.loop(0, steps // 2)
        def _(p):
            s0 = 2 * p
            start_gather(s0 + 1, 1, not_first=p > 0)
            process(s0, 0)

            @pl.when(s0 + 2 < steps)
            def _():
                start_gather(s0 + 2, 0, not_first=True)

            process(s0 + 1, 1)

        for slot, s in ((0, steps - 2), (1, steps - 1)):
            off = base + s * _W
            pltpu.make_async_copy(
                rows[slot], out_hbm.at[pl.ds(off, _W)], osem[slot]
            ).wait()

    g2 = gather_kernel(tabf, fused)
    g3 = g2.reshape(batch, seq, 2 * hidden)
    cond = in_left.reshape(batch, seq, 1)
    return jnp.where(cond, g3[:, :, :hidden], g3[:, :, hidden:])


# speedup vs baseline: 1.1252x; 1.1252x over previous
import dataclasses

import jax
import jax.numpy as jnp
from jax import lax
from jax.experimental import pallas as pl
from jax.experimental.pallas import tpu as pltpu
from jax.experimental.pallas import tpu_sc as plsc

_CP = pltpu.CompilerParams()
if "needs_layout_passes" in pltpu.CompilerParams.__dataclass_fields__:
    _CP = dataclasses.replace(_CP, needs_layout_passes=False)

_NUM_CORES = 2
_NUM_SUBCORES = 16
_NUM_WORKERS = _NUM_CORES * _NUM_SUBCORES
_W = 400
_LANES = 16
_VB = 2048


_HALF = 512000


def _fuse_table(table):
    vocab, hidden = table.shape
    tt = table.T
    nblk = _HALF // _VB
    last_blk = (vocab - 1) // _VB

    def body(t1_ref, t2_ref, out_ref):
        out_ref[...] = jnp.concatenate(
            [t1_ref[...].T, t2_ref[...].T], axis=1
        )

    return pl.pallas_call(
        body,
        grid=(nblk,),
        in_specs=[
            pl.BlockSpec((hidden, _VB), lambda i: (0, i)),
            pl.BlockSpec(
                (hidden, _VB), lambda i: (0, jnp.minimum(nblk + i, last_blk))
            ),
        ],
        out_specs=pl.BlockSpec((_VB, 2 * hidden), lambda i: (i, 0)),
        out_shape=jax.ShapeDtypeStruct((_HALF, 2 * hidden), table.dtype),
    )(tt, tt)


def kernel(input_ids, table):
    batch, seq = input_ids.shape
    vocab, hidden = table.shape
    tabf = _fuse_table(table)
    ids_t = input_ids.T
    ids = ids_t.reshape(n_flat := batch * seq)
    in_left = ids < _HALF
    fused = jnp.where(in_left, ids, ids - _HALF).astype(jnp.int32)

    per_worker = n_flat // _NUM_WORKERS
    steps = per_worker // _W
    assert per_worker % _W == 0 and n_flat % _NUM_WORKERS == 0

    mesh = plsc.VectorSubcoreMesh(core_axis_name="c", subcore_axis_name="s")

    @pl.kernel(
        out_type=jax.ShapeDtypeStruct((n_flat, 2 * hidden), table.dtype),
        mesh=mesh,
        scratch_types=[
            pltpu.VMEM((_W,), jnp.int32),
            pltpu.VMEM((_W,), jnp.int32),
            pltpu.VMEM((_W, 2 * hidden), table.dtype),
            pltpu.VMEM((_W, 2 * hidden), table.dtype),
            pltpu.SemaphoreType.DMA,
            pltpu.SemaphoreType.DMA,
            pltpu.SemaphoreType.DMA,
            pltpu.SemaphoreType.DMA,
        ],
        compiler_params=_CP,
    )
    def gather_kernel(tab_hbm, fused_hbm, out_hbm, fidx0, fidx1,
                      rows0, rows1, gsem0, gsem1, osem0, osem1):
        wid = lax.axis_index("s") * _NUM_CORES + lax.axis_index("c")
        base = wid * per_worker
        fidx = (fidx0, fidx1)
        rows = (rows0, rows1)
        gsem = (gsem0, gsem1)
        osem = (osem0, osem1)

        def start_gather(s, slot, not_first):
            off = base + s * _W
            poff = base + (s - 2) * _W

            @pl.when(not_first)
            def _():
                pltpu.make_async_copy(
                    rows[slot], out_hbm.at[pl.ds(poff, _W)], osem[slot]
                ).wait()

            pltpu.sync_copy(fused_hbm.at[pl.ds(off, _W)], fidx[slot])
            pltpu.make_async_copy(
                tab_hbm.at[fidx[slot]], rows[slot], gsem[slot]
            ).start()

        def process(s, slot):
            pltpu.make_async_copy(
                tab_hbm.at[fidx[slot]], rows[slot], gsem[slot]
            ).wait()
            off = base + s * _W
            pltpu.make_async_copy(
                rows[slot], out_hbm.at[pl.ds(off, _W)], osem[slot]
            ).start()

        start_gather(0, 0, not_first=False)

        @pl.loop(0, steps // 2)
        def _(p):
            s0 = 2 * p
            start_gather(s0 + 1, 1, not_first=p > 0)
            process(s0, 0)

            @pl.when(s0 + 2 < steps)
            def _():
                start_gather(s0 + 2, 0, not_first=True)

            process(s0 + 1, 1)

        for slot, s in ((0, steps - 2), (1, steps - 1)):
            off = base + s * _W
            pltpu.make_async_copy(
                rows[slot], out_hbm.at[pl.ds(off, _W)], osem[slot]
            ).wait()

    g2 = gather_kernel(tabf, fused)
    g3 = g2.reshape(seq, batch, 2 * hidden)
    cond_t = (ids_t < _HALF).reshape(seq, 1, batch)

    bb = 512

    def sel_body(g_ref, c_ref, out_ref):
        x = g_ref[0]
        left = x[:, :hidden].T
        right = x[:, hidden:].T
        out_ref[0] = jnp.where(c_ref[0], left, right)

    out_t = pl.pallas_call(
        sel_body,
        grid=(seq, batch // bb),
        in_specs=[
            pl.BlockSpec((1, bb, 2 * hidden), lambda t, c: (t, c, 0)),
            pl.BlockSpec((1, 1, bb), lambda t, c: (t, 0, c)),
        ],
        out_specs=pl.BlockSpec((1, hidden, bb), lambda t, c: (t, 0, c)),
        out_shape=jax.ShapeDtypeStruct((seq, hidden, batch), table.dtype),
    )(g3, cond_t)
    return out_t.transpose(2, 0, 1)


# speedup vs baseline: 1.8166x; 1.6144x over previous
import dataclasses

import jax
import jax.numpy as jnp
from jax import lax
from jax.experimental import pallas as pl
from jax.experimental.pallas import tpu as pltpu
from jax.experimental.pallas import tpu_sc as plsc

_CP = pltpu.CompilerParams()
if "needs_layout_passes" in pltpu.CompilerParams.__dataclass_fields__:
    _CP = dataclasses.replace(_CP, needs_layout_passes=False)

_NUM_CORES = 2
_NUM_SUBCORES = 16
_NUM_WORKERS = _NUM_CORES * _NUM_SUBCORES
_W = 400
_LANES = 16
_VB = 4096


_HALF = 512000


def _fuse_table(table):
    vocab, hidden = table.shape
    tt = table.T
    nblk = _HALF // _VB
    last_blk = (vocab - 1) // _VB

    def body(t1_ref, t2_ref, out_ref):
        out_ref[...] = jnp.concatenate(
            [t1_ref[...].T, t2_ref[...].T], axis=1
        )

    return pl.pallas_call(
        body,
        grid=(nblk,),
        in_specs=[
            pl.BlockSpec((hidden, _VB), lambda i: (0, i)),
            pl.BlockSpec(
                (hidden, _VB), lambda i: (0, jnp.minimum(nblk + i, last_blk))
            ),
        ],
        out_specs=pl.BlockSpec((_VB, 2 * hidden), lambda i: (i, 0)),
        out_shape=jax.ShapeDtypeStruct((_HALF, 2 * hidden), table.dtype),
    )(tt, tt)


def kernel(input_ids, table):
    batch, seq = input_ids.shape
    vocab, hidden = table.shape
    tabf = _fuse_table(table)
    ids_t = input_ids.T
    ids = ids_t.reshape(n_flat := batch * seq)
    in_left = ids < _HALF
    fused = jnp.where(in_left, ids, ids - _HALF).astype(jnp.int32)

    per_worker = n_flat // _NUM_WORKERS
    steps = per_worker // _W
    assert per_worker % _W == 0 and n_flat % _NUM_WORKERS == 0

    mesh = plsc.VectorSubcoreMesh(core_axis_name="c", subcore_axis_name="s")

    @pl.kernel(
        out_type=jax.ShapeDtypeStruct((n_flat, 2 * hidden), table.dtype),
        mesh=mesh,
        scratch_types=[
            pltpu.VMEM((_W,), jnp.int32),
            pltpu.VMEM((_W,), jnp.int32),
            pltpu.VMEM((_W, 2 * hidden), table.dtype),
            pltpu.VMEM((_W, 2 * hidden), table.dtype),
            pltpu.SemaphoreType.DMA,
            pltpu.SemaphoreType.DMA,
            pltpu.SemaphoreType.DMA,
            pltpu.SemaphoreType.DMA,
        ],
        compiler_params=_CP,
    )
    def gather_kernel(tab_hbm, fused_hbm, out_hbm, fidx0, fidx1,
                      rows0, rows1, gsem0, gsem1, osem0, osem1):
        wid = lax.axis_index("s") * _NUM_CORES + lax.axis_index("c")
        base = wid * per_worker
        fidx = (fidx0, fidx1)
        rows = (rows0, rows1)
        gsem = (gsem0, gsem1)
        osem = (osem0, osem1)

        def start_gather(s, slot, not_first):
            off = base + s * _W
            poff = base + (s - 2) * _W

            @pl.when(not_first)
            def _():
                pltpu.make_async_copy(
                    rows[slot], out_hbm.at[pl.ds(poff, _W)], osem[slot]
                ).wait()

            pltpu.sync_copy(fused_hbm.at[pl.ds(off, _W)], fidx[slot])
            pltpu.make_async_copy(
                tab_hbm.at[fidx[slot]], rows[slot], gsem[slot]
            ).start()

        def process(s, slot):
            pltpu.make_async_copy(
                tab_hbm.at[fidx[slot]], rows[slot], gsem[slot]
            ).wait()
            off = base + s * _W
            pltpu.make_async_copy(
                rows[slot], out_hbm.at[pl.ds(off, _W)], osem[slot]
            ).start()

        start_gather(0, 0, not_first=False)

        @pl.loop(0, steps // 2)
        def _(p):
            s0 = 2 * p
            start_gather(s0 + 1, 1, not_first=p > 0)
            process(s0, 0)

            @pl.when(s0 + 2 < steps)
            def _():
                start_gather(s0 + 2, 0, not_first=True)

            process(s0 + 1, 1)

        for slot, s in ((0, steps - 2), (1, steps - 1)):
            off = base + s * _W
            pltpu.make_async_copy(
                rows[slot], out_hbm.at[pl.ds(off, _W)], osem[slot]
            ).wait()

    g2 = gather_kernel(tabf, fused)
    g3 = g2.reshape(seq, batch, 2 * hidden)
    cond_t = (ids_t < _HALF).reshape(seq, 1, batch)

    bb = 2048

    def sel_body(g_ref, c_ref, out_ref):
        x = g_ref[0]
        left = x[:, :hidden].T
        right = x[:, hidden:].T
        out_ref[0] = jnp.where(c_ref[0], left, right)

    out_t = pl.pallas_call(
        sel_body,
        grid=(seq, batch // bb),
        in_specs=[
            pl.BlockSpec((1, bb, 2 * hidden), lambda t, c: (t, c, 0)),
            pl.BlockSpec((1, 1, bb), lambda t, c: (t, 0, c)),
        ],
        out_specs=pl.BlockSpec((1, hidden, bb), lambda t, c: (t, 0, c)),
        out_shape=jax.ShapeDtypeStruct((seq, hidden, batch), table.dtype),
    )(g3, cond_t)
    return out_t.transpose(2, 0, 1)


# speedup vs baseline: 1.8190x; 1.0013x over previous
import dataclasses

import jax
import jax.numpy as jnp
from jax import lax
from jax.experimental import pallas as pl
from jax.experimental.pallas import tpu as pltpu
from jax.experimental.pallas import tpu_sc as plsc

_CP = pltpu.CompilerParams()
if "needs_layout_passes" in pltpu.CompilerParams.__dataclass_fields__:
    _CP = dataclasses.replace(_CP, needs_layout_passes=False)

_NUM_CORES = 2
_NUM_SUBCORES = 16
_NUM_WORKERS = _NUM_CORES * _NUM_SUBCORES
_W = 400
_LANES = 16
_VB = 4096


_HALF = 512000


def _fuse_table(table):
    vocab, hidden = table.shape
    tt = table.T
    nblk = _HALF // _VB
    last_blk = (vocab - 1) // _VB

    def body(t1_ref, t2_ref, out_ref):
        out_ref[...] = jnp.concatenate(
            [t1_ref[...].T, t2_ref[...].T], axis=1
        )

    return pl.pallas_call(
        body,
        grid=(nblk,),
        in_specs=[
            pl.BlockSpec((hidden, _VB), lambda i: (0, i)),
            pl.BlockSpec(
                (hidden, _VB), lambda i: (0, jnp.minimum(nblk + i, last_blk))
            ),
        ],
        out_specs=pl.BlockSpec((_VB, 2 * hidden), lambda i: (i, 0)),
        out_shape=jax.ShapeDtypeStruct((_HALF, 2 * hidden), table.dtype),
        compiler_params=pltpu.CompilerParams(
            dimension_semantics=("parallel",)
        ),
    )(tt, tt)


def kernel(input_ids, table):
    batch, seq = input_ids.shape
    vocab, hidden = table.shape
    tabf = _fuse_table(table)
    ids_t = input_ids.T
    ids = ids_t.reshape(n_flat := batch * seq)
    in_left = ids < _HALF
    fused = jnp.where(in_left, ids, ids - _HALF).astype(jnp.int32)

    per_worker = n_flat // _NUM_WORKERS
    steps = per_worker // _W
    assert per_worker % _W == 0 and n_flat % _NUM_WORKERS == 0

    mesh = plsc.VectorSubcoreMesh(core_axis_name="c", subcore_axis_name="s")

    @pl.kernel(
        out_type=jax.ShapeDtypeStruct((n_flat, 2 * hidden), table.dtype),
        mesh=mesh,
        scratch_types=[
            pltpu.VMEM((_W,), jnp.int32),
            pltpu.VMEM((_W,), jnp.int32),
            pltpu.VMEM((_W, 2 * hidden), table.dtype),
            pltpu.VMEM((_W, 2 * hidden), table.dtype),
            pltpu.SemaphoreType.DMA,
            pltpu.SemaphoreType.DMA,
            pltpu.SemaphoreType.DMA,
            pltpu.SemaphoreType.DMA,
        ],
        compiler_params=_CP,
    )
    def gather_kernel(tab_hbm, fused_hbm, out_hbm, fidx0, fidx1,
                      rows0, rows1, gsem0, gsem1, osem0, osem1):
        wid = lax.axis_index("s") * _NUM_CORES + lax.axis_index("c")
        base = wid * per_worker
        fidx = (fidx0, fidx1)
        rows = (rows0, rows1)
        gsem = (gsem0, gsem1)
        osem = (osem0, osem1)

        def start_gather(s, slot, not_first):
            off = base + s * _W
            poff = base + (s - 2) * _W

            @pl.when(not_first)
            def _():
                pltpu.make_async_copy(
                    rows[slot], out_hbm.at[pl.ds(poff, _W)], osem[slot]
                ).wait()

            pltpu.sync_copy(fused_hbm.at[pl.ds(off, _W)], fidx[slot])
            pltpu.make_async_copy(
                tab_hbm.at[fidx[slot]], rows[slot], gsem[slot]
            ).start()

        def process(s, slot):
            pltpu.make_async_copy(
                tab_hbm.at[fidx[slot]], rows[slot], gsem[slot]
            ).wait()
            off = base + s * _W
            pltpu.make_async_copy(
                rows[slot], out_hbm.at[pl.ds(off, _W)], osem[slot]
            ).start()

        start_gather(0, 0, not_first=False)

        @pl.loop(0, steps // 2)
        def _(p):
            s0 = 2 * p
            start_gather(s0 + 1, 1, not_first=p > 0)
            process(s0, 0)

            @pl.when(s0 + 2 < steps)
            def _():
                start_gather(s0 + 2, 0, not_first=True)

            process(s0 + 1, 1)

        for slot, s in ((0, steps - 2), (1, steps - 1)):
            off = base + s * _W
            pltpu.make_async_copy(
                rows[slot], out_hbm.at[pl.ds(off, _W)], osem[slot]
            ).wait()

    g2 = gather_kernel(tabf, fused)
    g3 = g2.reshape(seq, batch, 2 * hidden)
    cond_t = (ids_t < _HALF).reshape(seq, 1, batch)

    bb = 2048

    def sel_body(g_ref, c_ref, out_ref):
        x = g_ref[0]
        left = x[:, :hidden].T
        right = x[:, hidden:].T
        out_ref[0] = jnp.where(c_ref[0], left, right)

    out_t = pl.pallas_call(
        sel_body,
        grid=(seq, batch // bb),
        in_specs=[
            pl.BlockSpec((1, bb, 2 * hidden), lambda t, c: (t, c, 0)),
            pl.BlockSpec((1, 1, bb), lambda t, c: (t, 0, c)),
        ],
        out_specs=pl.BlockSpec((1, hidden, bb), lambda t, c: (t, 0, c)),
        out_shape=jax.ShapeDtypeStruct((seq, hidden, batch), table.dtype),
        compiler_params=pltpu.CompilerParams(
            dimension_semantics=("parallel", "parallel")
        ),
    )(g3, cond_t)
    return out_t.transpose(2, 0, 1)


# speedup vs baseline: 2.0220x; 1.1116x over previous
import dataclasses

import jax
import jax.numpy as jnp
from jax import lax
from jax.experimental import pallas as pl
from jax.experimental.pallas import tpu as pltpu
from jax.experimental.pallas import tpu_sc as plsc

_CP = pltpu.CompilerParams()
if "needs_layout_passes" in pltpu.CompilerParams.__dataclass_fields__:
    _CP = dataclasses.replace(_CP, needs_layout_passes=False)

_NUM_CORES = 2
_NUM_SUBCORES = 16
_NUM_WORKERS = _NUM_CORES * _NUM_SUBCORES
_W = 400
_LANES = 16
_VB = 4096


_HALF = 512000


def _fuse_table(table):
    vocab, hidden = table.shape
    tt = table.T
    nblk = _HALF // _VB
    last_blk = (vocab - 1) // _VB

    def body(t1_ref, t2_ref, out_ref):
        out_ref[...] = jnp.concatenate(
            [t1_ref[...].T, t2_ref[...].T], axis=1
        )

    return pl.pallas_call(
        body,
        grid=(nblk,),
        in_specs=[
            pl.BlockSpec((hidden, _VB), lambda i: (0, i)),
            pl.BlockSpec(
                (hidden, _VB), lambda i: (0, jnp.minimum(nblk + i, last_blk))
            ),
        ],
        out_specs=pl.BlockSpec((_VB, 2 * hidden), lambda i: (i, 0)),
        out_shape=jax.ShapeDtypeStruct((_HALF, 2 * hidden), table.dtype),
        compiler_params=pltpu.CompilerParams(
            dimension_semantics=("parallel",)
        ),
    )(tt, tt)


def kernel(input_ids, table):
    batch, seq = input_ids.shape
    vocab, hidden = table.shape
    tabf = _fuse_table(table)
    ids_t = input_ids.T
    ids = ids_t.reshape(n_flat := batch * seq)
    in_left = ids < _HALF
    fused = jnp.where(in_left, ids, ids - _HALF).astype(jnp.int32)

    per_worker = n_flat // _NUM_WORKERS
    steps = per_worker // _W
    assert per_worker % _W == 0 and n_flat % _NUM_WORKERS == 0

    mesh = plsc.VectorSubcoreMesh(core_axis_name="c", subcore_axis_name="s")

    @pl.kernel(
        out_type=jax.ShapeDtypeStruct((n_flat, 2 * hidden), table.dtype),
        mesh=mesh,
        scratch_types=[
            pltpu.VMEM((_W,), jnp.int32),
            pltpu.VMEM((_W,), jnp.int32),
            pltpu.VMEM((_W, 2 * hidden), table.dtype),
            pltpu.VMEM((_W, 2 * hidden), table.dtype),
            pltpu.SemaphoreType.DMA,
            pltpu.SemaphoreType.DMA,
            pltpu.SemaphoreType.DMA,
            pltpu.SemaphoreType.DMA,
        ],
        compiler_params=_CP,
    )
    def gather_kernel(tab_hbm, fused_hbm, out_hbm, fidx0, fidx1,
                      rows0, rows1, gsem0, gsem1, osem0, osem1):
        wid = lax.axis_index("s") * _NUM_CORES + lax.axis_index("c")
        base = wid * per_worker
        fidx = (fidx0, fidx1)
        rows = (rows0, rows1)
        gsem = (gsem0, gsem1)
        osem = (osem0, osem1)

        def start_gather(s, slot, not_first):
            off = base + s * _W
            poff = base + (s - 2) * _W

            @pl.when(not_first)
            def _():
                pltpu.make_async_copy(
                    rows[slot], out_hbm.at[pl.ds(poff, _W)], osem[slot]
                ).wait()

            pltpu.sync_copy(fused_hbm.at[pl.ds(off, _W)], fidx[slot])
            pltpu.make_async_copy(
                tab_hbm.at[fidx[slot]], rows[slot], gsem[slot]
            ).start()

        def process(s, slot):
            pltpu.make_async_copy(
                tab_hbm.at[fidx[slot]], rows[slot], gsem[slot]
            ).wait()
            off = base + s * _W
            pltpu.make_async_copy(
                rows[slot], out_hbm.at[pl.ds(off, _W)], osem[slot]
            ).start()

        start_gather(0, 0, not_first=False)

        @pl.loop(0, steps // 2)
        def _(p):
            s0 = 2 * p
            start_gather(s0 + 1, 1, not_first=p > 0)
            process(s0, 0)

            @pl.when(s0 + 2 < steps)
            def _():
                start_gather(s0 + 2, 0, not_first=True)

            process(s0 + 1, 1)

        for slot, s in ((0, steps - 2), (1, steps - 1)):
            off = base + s * _W
            pltpu.make_async_copy(
                rows[slot], out_hbm.at[pl.ds(off, _W)], osem[slot]
            ).wait()

    g2 = gather_kernel(tabf, fused)
    g3 = g2.reshape(seq, batch, 2 * hidden)
    cond_t = (ids_t < _HALF).reshape(seq, 1, batch)

    bb = 4096

    def sel_body(g_ref, c_ref, out_ref):
        x = g_ref[0]
        left = x[:, :hidden].T
        right = x[:, hidden:].T
        out_ref[0] = jnp.where(c_ref[0], left, right)

    out_t = pl.pallas_call(
        sel_body,
        grid=(seq, batch // bb),
        in_specs=[
            pl.BlockSpec((1, bb, 2 * hidden), lambda t, c: (t, c, 0)),
            pl.BlockSpec((1, 1, bb), lambda t, c: (t, 0, c)),
        ],
        out_specs=pl.BlockSpec((1, hidden, bb), lambda t, c: (t, 0, c)),
        out_shape=jax.ShapeDtypeStruct((seq, hidden, batch), table.dtype),
        compiler_params=pltpu.CompilerParams(
            dimension_semantics=("parallel", "parallel")
        ),
    )(g3, cond_t)
    return out_t.transpose(2, 0, 1)


# speedup vs baseline: 2.1298x; 1.0534x over previous
import dataclasses

import jax
import jax.numpy as jnp
from jax import lax
from jax.experimental import pallas as pl
from jax.experimental.pallas import tpu as pltpu
from jax.experimental.pallas import tpu_sc as plsc

_CP = pltpu.CompilerParams()
if "needs_layout_passes" in pltpu.CompilerParams.__dataclass_fields__:
    _CP = dataclasses.replace(_CP, needs_layout_passes=False)

_NUM_CORES = 2
_NUM_SUBCORES = 16
_NUM_WORKERS = _NUM_CORES * _NUM_SUBCORES
_W = 400
_LANES = 16
_VB = 4096


_HALF = 512000


def _fuse_table(table):
    vocab, hidden = table.shape
    tt = table.T
    nblk = _HALF // _VB
    last_blk = (vocab - 1) // _VB

    def body(t1_ref, t2_ref, out_ref):
        out_ref[...] = jnp.concatenate(
            [t1_ref[...].T, t2_ref[...].T], axis=1
        )

    return pl.pallas_call(
        body,
        grid=(nblk,),
        in_specs=[
            pl.BlockSpec((hidden, _VB), lambda i: (0, i)),
            pl.BlockSpec(
                (hidden, _VB), lambda i: (0, jnp.minimum(nblk + i, last_blk))
            ),
        ],
        out_specs=pl.BlockSpec((_VB, 2 * hidden), lambda i: (i, 0)),
        out_shape=jax.ShapeDtypeStruct((_HALF, 2 * hidden), table.dtype),
        compiler_params=pltpu.CompilerParams(
            dimension_semantics=("parallel",)
        ),
    )(tt, tt)


def kernel(input_ids, table):
    batch, seq = input_ids.shape
    vocab, hidden = table.shape
    tabf = _fuse_table(table)
    ids_t = input_ids.T
    ids = ids_t.reshape(n_flat := batch * seq)
    in_left = ids < _HALF
    fused = jnp.where(in_left, ids, ids - _HALF).astype(jnp.int32)

    per_worker = n_flat // _NUM_WORKERS
    steps = per_worker // _W
    assert per_worker % _W == 0 and n_flat % _NUM_WORKERS == 0

    mesh = plsc.VectorSubcoreMesh(core_axis_name="c", subcore_axis_name="s")

    @pl.kernel(
        out_type=jax.ShapeDtypeStruct((n_flat, 2 * hidden), table.dtype),
        mesh=mesh,
        scratch_types=[
            pltpu.VMEM((_W,), jnp.int32),
            pltpu.VMEM((_W,), jnp.int32),
            pltpu.VMEM((_W, 2 * hidden), table.dtype),
            pltpu.VMEM((_W, 2 * hidden), table.dtype),
            pltpu.SemaphoreType.DMA,
            pltpu.SemaphoreType.DMA,
            pltpu.SemaphoreType.DMA,
            pltpu.SemaphoreType.DMA,
        ],
        compiler_params=_CP,
    )
    def gather_kernel(tab_hbm, fused_hbm, out_hbm, fidx0, fidx1,
                      rows0, rows1, gsem0, gsem1, osem0, osem1):
        wid = lax.axis_index("s") * _NUM_CORES + lax.axis_index("c")
        base = wid * per_worker
        fidx = (fidx0, fidx1)
        rows = (rows0, rows1)
        gsem = (gsem0, gsem1)
        osem = (osem0, osem1)

        def start_gather(s, slot, not_first):
            off = base + s * _W
            poff = base + (s - 2) * _W

            @pl.when(not_first)
            def _():
                pltpu.make_async_copy(
                    rows[slot], out_hbm.at[pl.ds(poff, _W)], osem[slot]
                ).wait()

            pltpu.sync_copy(fused_hbm.at[pl.ds(off, _W)], fidx[slot])
            pltpu.make_async_copy(
                tab_hbm.at[fidx[slot]], rows[slot], gsem[slot]
            ).start()

        def process(s, slot):
            pltpu.make_async_copy(
                tab_hbm.at[fidx[slot]], rows[slot], gsem[slot]
            ).wait()
            off = base + s * _W
            pltpu.make_async_copy(
                rows[slot], out_hbm.at[pl.ds(off, _W)], osem[slot]
            ).start()

        start_gather(0, 0, not_first=False)

        @pl.loop(0, steps // 2)
        def _(p):
            s0 = 2 * p
            start_gather(s0 + 1, 1, not_first=p > 0)
            process(s0, 0)

            @pl.when(s0 + 2 < steps)
            def _():
                start_gather(s0 + 2, 0, not_first=True)

            process(s0 + 1, 1)

        for slot, s in ((0, steps - 2), (1, steps - 1)):
            off = base + s * _W
            pltpu.make_async_copy(
                rows[slot], out_hbm.at[pl.ds(off, _W)], osem[slot]
            ).wait()

    g2 = gather_kernel(tabf, fused)
    g3 = g2.reshape(seq, batch, 2 * hidden)
    cond_t = (ids_t < _HALF).reshape(seq, 1, batch)

    bb = 4096

    tb = 4

    def sel_body(g_ref, c_ref, out_ref):
        for tt in range(tb):
            x = g_ref[tt]
            left = x[:, :hidden].T
            right = x[:, hidden:].T
            out_ref[tt] = jnp.where(c_ref[tt], left, right)

    out_t = pl.pallas_call(
        sel_body,
        grid=(seq // tb,),
        in_specs=[
            pl.BlockSpec((tb, bb, 2 * hidden), lambda t: (t, 0, 0)),
            pl.BlockSpec((tb, 1, bb), lambda t: (t, 0, 0)),
        ],
        out_specs=pl.BlockSpec((tb, hidden, bb), lambda t: (t, 0, 0)),
        out_shape=jax.ShapeDtypeStruct((seq, hidden, batch), table.dtype),
        compiler_params=pltpu.CompilerParams(
            dimension_semantics=("parallel",)
        ),
    )(g3, cond_t)
    return out_t.transpose(2, 0, 1)
